# Initial kernel scaffold; baseline (speedup 1.0000x reference)
import jax
import jax.numpy as jnp
from jax.experimental import pallas as pl


def kernel(idx, emb_weight):
    raise NotImplementedError("write your pallas kernel here")


# speedup vs baseline: 3.3696x; 3.3696x over previous
import functools

import jax
import jax.numpy as jnp
from jax import lax
from jax.experimental import pallas as pl
from jax.experimental.pallas import tpu as pltpu
from jax.experimental.pallas import tpu_sc as plsc

NUM_EMBEDDINGS = 100000
OUT_DIM = 128
BATCH = 16384
FIELDS = 26

NC = 2
NS = 16
NW = NC * NS

B_TOTAL = BATCH * FIELDS
B_PER_W = B_TOTAL // NW
CHUNK = 128
NCHUNK = B_PER_W // CHUNK
NBUF = 4
NROUNDS = NCHUNK // NBUF

assert B_PER_W * NW == B_TOTAL
assert CHUNK * NCHUNK == B_PER_W
assert NBUF * NROUNDS == NCHUNK


def _sc_gather(idx_grouped, emb_weight):
    mesh = plsc.VectorSubcoreMesh(
        core_axis_name="c", subcore_axis_name="s", num_cores=NC, num_subcores=NS
    )

    @functools.partial(
        pl.kernel,
        out_type=jax.ShapeDtypeStruct((B_TOTAL, OUT_DIM), jnp.float32),
        mesh=mesh,
        scratch_types=[
            pltpu.VMEM((NCHUNK, CHUNK), jnp.int32),
            pltpu.VMEM((NBUF, CHUNK, OUT_DIM), jnp.float32),
        ]
        + [pltpu.SemaphoreType.DMA] * (2 * NBUF),
    )
    def k(idx_hbm, table_hbm, out_hbm, idx_v, rows_v, *sems):
        gsems = sems[:NBUF]
        ssems = sems[NBUF:]
        wid = lax.axis_index("s") * NC + lax.axis_index("c")
        base = wid * B_PER_W

        pltpu.sync_copy(idx_hbm.at[wid], idx_v)

        def gather(chunk, k_buf):
            return pltpu.make_async_copy(
                table_hbm.at[idx_v.at[chunk]], rows_v.at[k_buf], gsems[k_buf]
            )

        def store(chunk, k_buf):
            return pltpu.make_async_copy(
                rows_v.at[k_buf],
                out_hbm.at[pl.ds(base + chunk * CHUNK, CHUNK)],
                ssems[k_buf],
            )

        for kb in range(NBUF):
            gather(kb, kb).start()

        @pl.loop(0, NROUNDS)
        def _(r):
            c0 = r * NBUF
            for kb in range(NBUF):
                gather(c0 + kb, kb).wait()
                store(c0 + kb, kb).start()
            for kb in range(NBUF):
                store(c0 + kb, kb).wait()

                @pl.when(r < NROUNDS - 1)
                def _():
                    gather(c0 + NBUF + kb, kb).start()

    return k(idx_grouped, emb_weight)


@jax.jit
def kernel(idx, emb_weight):
    idx_grouped = idx.reshape(NW, NCHUNK, CHUNK).astype(jnp.int32)
    out = _sc_gather(idx_grouped, emb_weight)
    return out.reshape(BATCH, FIELDS, OUT_DIM)


# speedup vs baseline: 5.6538x; 1.6779x over previous
import functools

import jax
import jax.numpy as jnp
from jax import lax
from jax.experimental import pallas as pl
from jax.experimental.pallas import tpu as pltpu
from jax.experimental.pallas import tpu_sc as plsc

NUM_EMBEDDINGS = 100000
OUT_DIM = 128
BATCH = 16384
FIELDS = 26

NC = 2
NS = 16
NW = NC * NS

NB = BATCH // NW
NBC = 4
ROWS = NBC * FIELDS
NCHUNK = NB // NBC
NBUF = 4
NROUNDS = NCHUNK // NBUF

assert NB * NW == BATCH
assert NBC * NCHUNK == NB
assert NBUF * NROUNDS == NCHUNK


def _sc_gather(idx_grouped, emb_weight):
    mesh = plsc.VectorSubcoreMesh(
        core_axis_name="c", subcore_axis_name="s", num_cores=NC, num_subcores=NS
    )

    @functools.partial(
        pl.kernel,
        out_type=jax.ShapeDtypeStruct((BATCH, FIELDS, OUT_DIM), jnp.float32),
        mesh=mesh,
        scratch_types=[
            pltpu.VMEM((NCHUNK, ROWS), jnp.int32),
            pltpu.VMEM((NBUF, ROWS, OUT_DIM), jnp.float32),
        ]
        + [pltpu.SemaphoreType.DMA] * (2 * NBUF),
        compiler_params=pltpu.CompilerParams(use_tc_tiling_on_sc=True),
    )
    def k(idx_hbm, table_hbm, out_hbm, idx_v, rows_v, *sems):
        gsems = sems[:NBUF]
        ssems = sems[NBUF:]
        wid = lax.axis_index("s") * NC + lax.axis_index("c")
        b0 = wid * NB

        pltpu.sync_copy(idx_hbm.at[wid], idx_v)

        def gather(chunk, kb):
            return pltpu.make_async_copy(
                table_hbm.at[idx_v.at[chunk]], rows_v.at[kb], gsems[kb]
            )

        def store(chunk, kb, j):
            return pltpu.make_async_copy(
                rows_v.at[kb, pl.ds(j * FIELDS, FIELDS)],
                out_hbm.at[b0 + chunk * NBC + j],
                ssems[kb],
            )

        for kb in range(NBUF):
            gather(kb, kb).start()

        @pl.loop(0, NROUNDS)
        def _(r):
            c0 = r * NBUF
            for kb in range(NBUF):
                gather(c0 + kb, kb).wait()
                for j in range(NBC):
                    store(c0 + kb, kb, j).start()
            for kb in range(NBUF):
                for j in range(NBC):
                    store(c0 + kb, kb, j).wait()

                @pl.when(r < NROUNDS - 1)
                def _():
                    gather(c0 + NBUF + kb, kb).start()

    return k(idx_grouped, emb_weight)


@jax.jit
def kernel(idx, emb_weight):
    idx_grouped = idx.reshape(NW, NCHUNK, ROWS).astype(jnp.int32)
    return _sc_gather(idx_grouped, emb_weight)


# speedup vs baseline: 11.5186x; 2.0373x over previous
import functools

import jax
import jax.numpy as jnp
from jax import lax
from jax.experimental import pallas as pl
from jax.experimental.pallas import tpu as pltpu
from jax.experimental.pallas import tpu_sc as plsc

NUM_EMBEDDINGS = 100000
OUT_DIM = 128
BATCH = 16384
FIELDS = 26

NC = 2
NS = 16
NW = NC * NS

NB = BATCH // NW
CH = 128
NBUF = NB // CH

assert NB * NW == BATCH
assert NBUF * CH == NB


def _sc_gather(idx_grouped, emb_weight):
    mesh = plsc.VectorSubcoreMesh(
        core_axis_name="c", subcore_axis_name="s", num_cores=NC, num_subcores=NS
    )

    @functools.partial(
        pl.kernel,
        out_type=jax.ShapeDtypeStruct((FIELDS, BATCH, OUT_DIM), jnp.float32),
        mesh=mesh,
        scratch_types=[
            pltpu.VMEM((FIELDS, NB), jnp.int32),
            pltpu.VMEM((NBUF, CH, OUT_DIM), jnp.float32),
        ]
        + [pltpu.SemaphoreType.DMA] * (2 * NBUF),
    )
    def k(idx_hbm, table_hbm, out_hbm, idx_v, rows_v, *sems):
        gsems = sems[:NBUF]
        ssems = sems[NBUF:]
        wid = lax.axis_index("s") * NC + lax.axis_index("c")
        b0 = wid * NB

        pltpu.sync_copy(idx_hbm.at[wid], idx_v)

        def gather(f, kb):
            return pltpu.make_async_copy(
                table_hbm.at[idx_v.at[f, pl.ds(kb * CH, CH)]],
                rows_v.at[kb],
                gsems[kb],
            )

        def store(f, kb):
            return pltpu.make_async_copy(
                rows_v.at[kb],
                out_hbm.at[f, pl.ds(b0 + kb * CH, CH)],
                ssems[kb],
            )

        for kb in range(NBUF):
            gather(0, kb).start()

        @pl.loop(0, FIELDS)
        def _(f):
            for kb in range(NBUF):
                gather(f, kb).wait()
                store(f, kb).start()
            for kb in range(NBUF):
                store(f, kb).wait()

                @pl.when(f < FIELDS - 1)
                def _():
                    gather(f + 1, kb).start()

    return k(idx_grouped, emb_weight)


@jax.jit
def kernel(idx, emb_weight):
    idx_grouped = (
        idx.astype(jnp.int32).T.reshape(FIELDS, NW, NB).transpose(1, 0, 2)
    )
    out_fmajor = _sc_gather(idx_grouped, emb_weight)
    return out_fmajor.transpose(1, 0, 2)


# speedup vs baseline: 11.8244x; 1.0265x over previous
import functools

import jax
import jax.numpy as jnp
from jax import lax
from jax.experimental import pallas as pl
from jax.experimental.pallas import tpu as pltpu
from jax.experimental.pallas import tpu_sc as plsc

NUM_EMBEDDINGS = 100000
OUT_DIM = 128
BATCH = 16384
FIELDS = 26

NC = 2
NS = 16
NW = NC * NS

NB = BATCH // NW
CH = 64
CPF = NB // CH
NBUF = 8
DIST = 4
NCHUNK = FIELDS * CPF

assert NB * NW == BATCH
assert CPF == NBUF


def _sc_gather(idx_grouped, emb_weight):
    mesh = plsc.VectorSubcoreMesh(
        core_axis_name="c", subcore_axis_name="s", num_cores=NC, num_subcores=NS
    )

    @functools.partial(
        pl.kernel,
        out_type=jax.ShapeDtypeStruct((FIELDS, BATCH, OUT_DIM), jnp.float32),
        mesh=mesh,
        scratch_types=[
            pltpu.VMEM((FIELDS, NB), jnp.int32),
            pltpu.VMEM((NBUF, CH, OUT_DIM), jnp.float32),
        ]
        + [pltpu.SemaphoreType.DMA] * (2 * NBUF),
    )
    def k(idx_hbm, table_hbm, out_hbm, idx_v, rows_v, *sems):
        gsems = sems[:NBUF]
        ssems = sems[NBUF:]
        wid = lax.axis_index("s") * NC + lax.axis_index("c")
        b0 = wid * NB

        pltpu.sync_copy(idx_hbm.at[wid], idx_v)

        def gather(f, cb, kb):
            return pltpu.make_async_copy(
                table_hbm.at[idx_v.at[f, pl.ds(cb * CH, CH)]],
                rows_v.at[kb],
                gsems[kb],
            )

        def store(f, cb, kb):
            return pltpu.make_async_copy(
                rows_v.at[kb],
                out_hbm.at[f, pl.ds(b0 + cb * CH, CH)],
                ssems[kb],
            )

        for kb in range(DIST):
            gather(0, kb, kb).start()

        @pl.loop(0, FIELDS)
        def _(r):
            for k in range(NBUF):
                kp = (k + DIST) % NBUF
                if k < NBUF - DIST:
                    @pl.when(r > 0)
                    def _():
                        store(r - 1, kp, kp).wait()

                    gather(r, kp, kp).start()
                else:
                    store(r, kp, kp).wait()

                    @pl.when(r < FIELDS - 1)
                    def _():
                        gather(r + 1, kp, kp).start()

                gather(r, k, k).wait()
                store(r, k, k).start()

        for kb in range(DIST, NBUF):
            store(FIELDS - 1, kb, kb).wait()

    return k(idx_grouped, emb_weight)


@jax.jit
def kernel(idx, emb_weight):
    idx_grouped = (
        idx.astype(jnp.int32).T.reshape(FIELDS, NW, NB).transpose(1, 0, 2)
    )
    out_fmajor = _sc_gather(idx_grouped, emb_weight)
    return out_fmajor.transpose(1, 0, 2)
